# Initial kernel scaffold; baseline (speedup 1.0000x reference)
import functools

import jax
import jax.numpy as jnp
from jax import lax
from jax.experimental import pallas as pl
from jax.experimental.pallas import tpu as pltpu
from jax.experimental.pallas import tpu_sc as plsc

N = 50000
H = 64
E = 800000
NW = 32
NTILE = 1568
NPAD = NTILE * NW
NSC = NTILE * 16
CHUNK = 4000
NCHUNKS = E // CHUNK
GRP = CHUNK // 16
BATCH = 64
CAP = CHUNK + BATCH + 32
RB = 2000
NB = N // RB
NEG = float(-3.0e38)


def _lk(x):
    return jnp.where(x >= 0, x, 0.1 * x)


def _dot(a, b):
    return jax.lax.dot_general(a, b, (((1,), (0,)), ((), ())),
                               precision=jax.lax.Precision.HIGHEST)


def _res(x, w1, b1, w2, b2):
    return x + _dot(_lk(_dot(x, w1) + b1), w2) + b2


_sc_mesh = plsc.VectorSubcoreMesh(core_axis_name="c", subcore_axis_name="s")


@functools.partial(
    pl.kernel,
    mesh=_sc_mesh,
    out_type=[
        jax.ShapeDtypeStruct((NPAD, H), jnp.float32),
        jax.ShapeDtypeStruct((NPAD, H), jnp.float32),
        jax.ShapeDtypeStruct((NPAD,), jnp.float32),
    ],
    scratch_types=[
        pltpu.VMEM((NTILE, H), jnp.float32),
        pltpu.VMEM((CHUNK,), jnp.int32),
        pltpu.VMEM((CHUNK,), jnp.int32),
        pltpu.VMEM((CAP,), jnp.int32),
        pltpu.VMEM((CAP,), jnp.int32),
        pltpu.VMEM((1, BATCH), jnp.int32),
        pltpu.VMEM((BATCH, H), jnp.float32),
        pltpu.VMEM((BATCH,), jnp.float32),
        pltpu.VMEM((NTILE,), jnp.float32),
        pltpu.VMEM_SHARED((NSC + 16, H), jnp.float32),
        pltpu.VMEM_SHARED((NSC + 16,), jnp.float32),
        pltpu.SemaphoreType.DMA,
    ],
)
def _segment_kernel(h_hbm, src_hbm, dst_hbm, out_sum, out_max, out_deg,
                    macc, dstst, srcst, csrc, cdst, sidx2d, rows, ones,
                    zbuf, ssum, sdeg, sem):
    c = lax.axis_index("c")
    s = lax.axis_index("s")
    wid = c * 16 + s
    lo = wid * NTILE
    sc_base = c * NSC
    tloc = s * NTILE

    negv = jnp.full((16,), NEG, jnp.float32)
    zv = jnp.zeros((16,), jnp.float32)
    lanes = lax.iota(jnp.int32, 16)

    def _init_macc(r, carry):
        for j in range(4):
            macc[r, pl.ds(16 * j, 16)] = negv
        return carry
    lax.fori_loop(0, NTILE, _init_macc, 0)

    def _init_zbuf(t, carry):
        zbuf[pl.ds(t * 16, 16)] = zv
        return carry
    lax.fori_loop(0, NTILE // 16, _init_zbuf, 0)

    def _init_rows(r, carry):
        for j in range(4):
            rows[r, pl.ds(16 * j, 16)] = zv
        return carry
    lax.fori_loop(0, BATCH, _init_rows, 0)

    for g in range(BATCH // 16):
        ones[pl.ds(g * 16, 16)] = jnp.full((16,), 1.0, jnp.float32)

    for t in range(NTILE // BATCH):
        pltpu.sync_copy(rows, ssum.at[pl.ds(tloc + t * BATCH, BATCH)])
    pltpu.sync_copy(rows.at[pl.ds(0, NTILE - (NTILE // BATCH) * BATCH)],
                    ssum.at[pl.ds(tloc + (NTILE // BATCH) * BATCH,
                                  NTILE - (NTILE // BATCH) * BATCH)])
    pltpu.sync_copy(zbuf, sdeg.at[pl.ds(tloc, NTILE)])

    @pl.when(s == 0)
    def _zero_dump():
        pltpu.sync_copy(rows.at[pl.ds(0, 16)], ssum.at[pl.ds(NSC, 16)])
        pltpu.sync_copy(zbuf.at[pl.ds(0, 16)], sdeg.at[pl.ds(NSC, 16)])

    plsc.subcore_barrier()

    def _flush(off):
        for g in range(4):
            csrc[pl.ds(off + g * 16, 16)] = lanes + (g * 16)
            cdst[pl.ds(off + g * 16, 16)] = lanes + NSC
        nbatch = (off + BATCH - 1) // BATCH

        def _batch(bi, carry):
            base = bi * BATCH
            for g in range(4):
                sidx2d[0, pl.ds(g * 16, 16)] = cdst[pl.ds(base + g * 16, 16)]
            gidx = csrc.at[pl.ds(base, BATCH)]
            pltpu.async_copy(h_hbm.at[gidx], rows, sem).wait()
            nreal = jnp.minimum(off - base, BATCH)

            def _edge(i, ecarry):
                ld = cdst[base + i] - tloc
                for j in range(4):
                    rj = rows[i, pl.ds(16 * j, 16)]
                    mj = macc[ld, pl.ds(16 * j, 16)]
                    macc[ld, pl.ds(16 * j, 16)] = jnp.maximum(mj, rj)
                return ecarry
            lax.fori_loop(0, nreal, _edge, 0)

            pltpu.sync_copy(rows, ssum.at[sidx2d.at[0]], add=True)
            pltpu.sync_copy(ones, sdeg.at[sidx2d.at[0]], add=True)
            return carry
        lax.fori_loop(0, nbatch, _batch, 0)

    def _chunk(k, carry):
        pltpu.sync_copy(dst_hbm.at[pl.ds(k * CHUNK, CHUNK)], dstst)
        pltpu.sync_copy(src_hbm.at[pl.ds(k * CHUNK, CHUNK)], srcst)

        def _group(g, off):
            d = dstst[pl.ds(g * 16, 16)]
            m = (d >= lo) & (d < lo + NTILE)
            sv = srcst[pl.ds(g * 16, 16)]
            plsc.store_compressed(csrc.at[pl.ds(off, 16)], sv, m)
            plsc.store_compressed(cdst.at[pl.ds(off, 16)], d - sc_base, m)
            return off + jnp.sum(m.astype(jnp.int32))
        off = lax.fori_loop(0, GRP, _group, 0)
        _flush(off)
        return carry
    lax.fori_loop(0, NCHUNKS, _chunk, 0)

    plsc.subcore_barrier()
    pltpu.sync_copy(ssum.at[pl.ds(tloc, NTILE)], out_sum.at[pl.ds(lo, NTILE)])
    pltpu.sync_copy(sdeg.at[pl.ds(tloc, NTILE)], out_deg.at[pl.ds(lo, NTILE)])
    pltpu.sync_copy(macc, out_max.at[pl.ds(lo, NTILE)])


def _stage_a_body(re_ref, pe_ref, nW, nb_, hW, hb_, nm1W, nm1b, nm2W, nm2b,
                  hm1W, hm1b, hm2W, hm2b, pm1W, pm1b, pm2W, pm2b,
                  nf_ref, hp_ref, hs_ref):
    i = pl.program_id(0)
    nf = _dot(re_ref[...], nW[...]) + nb_[...]
    nf = _res(nf, nm1W[...], nm1b[...], nm2W[...], nm2b[...])
    hf = _dot(pe_ref[...], hW[...]) + hb_[...]
    hf = _res(hf, hm1W[...], hm1b[...], hm2W[...], hm2b[...])
    hp = _res(hf, pm1W[...], pm1b[...], pm2W[...], pm2b[...])
    nf_ref[...] = nf
    hp_ref[...] = hp
    bs = jnp.sum(hf, axis=0, keepdims=True)

    @pl.when(i == 0)
    def _():
        hs_ref[...] = bs

    @pl.when(i != 0)
    def _():
        hs_ref[...] = hs_ref[...] + bs


def _combine(nfv, hsv, hmv, dv, rWv, rbv):
    hmv = jnp.where(dv > 0, hmv, 0.0)
    hmean = hsv / jnp.maximum(dv, 1.0)
    t = (_dot(hsv, rWv[0:64]) + _dot(hmv, rWv[64:128])
         + _dot(hmean, rWv[128:192]) + _dot(nfv, rWv[192:256]) + rbv)
    return nfv + _lk(t)


def _stage_c_body(nf, hs, hm, dg, rW, rb_, cm1W, cm1b, cm2W, cm2b,
                  rf_ref, hc_ref):
    rf = _combine(nf[...], hs[...], hm[...], dg[...], rW[...], rb_[...])
    rf_ref[...] = rf
    hc_ref[...] = _res(rf, cm1W[...], cm1b[...], cm2W[...], cm2b[...])


def _stage_e_body(rf, hs, hm, dg, rW, rb_, hsum, w1, b1, w2, b2, w3, b3,
                  out_ref, acc_ref):
    i = pl.program_id(0)
    rf2 = _combine(rf[...], hs[...], hm[...], dg[...], rW[...], rb_[...])
    bs = jnp.sum(rf2, axis=0, keepdims=True)

    @pl.when(i == 0)
    def _():
        acc_ref[...] = bs

    @pl.when(i != 0)
    def _():
        acc_ref[...] = acc_ref[...] + bs

    @pl.when(i == NB - 1)
    def _():
        mh = hsum[...] * (1.0 / N)
        mr = acc_ref[...] * (1.0 / N)
        w1v = w1[...]
        h1 = _lk(_dot(mh, w1v[0:64]) + _dot(mr, w1v[64:128]) + b1[...])
        h2 = _lk(_dot(h1, w2[...]) + b2[...])
        out_ref[...] = _dot(h2, w3[...]) + b3[...]


def _full(shape):
    return pl.BlockSpec(shape, lambda i: tuple(0 for _ in shape))


def _rows(width):
    return pl.BlockSpec((RB, width), lambda i: (i, 0))


def kernel(router_embed, packet_embed, params, pass_edge_index,
           connect_edge_index):
    p = params
    f32 = jnp.float32

    def w(name):
        return p[name].astype(f32)

    def b2d(name):
        return p[name].astype(f32).reshape(1, -1)

    stage_a = pl.pallas_call(
        _stage_a_body,
        grid=(NB,),
        in_specs=[
            _rows(6), _rows(1),
            _full((6, H)), _full((1, H)), _full((1, H)), _full((1, H)),
            _full((H, H)), _full((1, H)), _full((H, H)), _full((1, H)),
            _full((H, H)), _full((1, H)), _full((H, H)), _full((1, H)),
            _full((H, H)), _full((1, H)), _full((H, H)), _full((1, H)),
        ],
        out_specs=[_rows(H), _rows(H), _full((1, H))],
        out_shape=[
            jax.ShapeDtypeStruct((N, H), f32),
            jax.ShapeDtypeStruct((N, H), f32),
            jax.ShapeDtypeStruct((1, H), f32),
        ],
    )
    nf, hp, hsum = stage_a(
        router_embed.astype(f32), packet_embed.astype(f32),
        w('fg_node_W'), b2d('fg_node_b'), w('fg_hyper_W'), b2d('fg_hyper_b'),
        w('fg_nmlp_W1'), b2d('fg_nmlp_b1'), w('fg_nmlp_W2'), b2d('fg_nmlp_b2'),
        w('fg_hmlp_W1'), b2d('fg_hmlp_b1'), w('fg_hmlp_W2'), b2d('fg_hmlp_b2'),
        w('pass_m_W1'), b2d('pass_m_b1'), w('pass_m_W2'), b2d('pass_m_b2'),
    )

    s1, m1, d1 = _segment_kernel(hp, pass_edge_index[0], pass_edge_index[1])

    stage_c = pl.pallas_call(
        _stage_c_body,
        grid=(NB,),
        in_specs=[
            _rows(H), _rows(H), _rows(H), _rows(1),
            _full((4 * H, H)), _full((1, H)),
            _full((H, H)), _full((1, H)), _full((H, H)), _full((1, H)),
        ],
        out_specs=[_rows(H), _rows(H)],
        out_shape=[
            jax.ShapeDtypeStruct((N, H), f32),
            jax.ShapeDtypeStruct((N, H), f32),
        ],
    )
    rf, hc = stage_c(
        nf, s1, m1, d1.reshape(NPAD, 1),
        w('pass_r_W'), b2d('pass_r_b'),
        w('conn_m_W1'), b2d('conn_m_b1'), w('conn_m_W2'), b2d('conn_m_b2'),
    )

    s2, m2, d2 = _segment_kernel(hc, connect_edge_index[0],
                                 connect_edge_index[1])

    stage_e = pl.pallas_call(
        _stage_e_body,
        grid=(NB,),
        in_specs=[
            _rows(H), _rows(H), _rows(H), _rows(1),
            _full((4 * H, H)), _full((1, H)), _full((1, H)),
            _full((2 * H, H)), _full((1, H)),
            _full((H, H)), _full((1, H)),
            _full((H, 2)), _full((1, 2)),
        ],
        out_specs=[_full((1, 2))],
        out_shape=[jax.ShapeDtypeStruct((1, 2), f32)],
        scratch_shapes=[pltpu.VMEM((1, H), f32)],
    )
    out = stage_e(
        rf, s2, m2, d2.reshape(NPAD, 1),
        w('conn_r_W'), b2d('conn_r_b'), hsum,
        w('head_W1'), b2d('head_b1'), w('head_W2'), b2d('head_b2'),
        w('head_W3'), b2d('head_b3'),
    )
    return out[0].reshape(2)


# speedup vs baseline: 1.4716x; 1.4716x over previous
import functools

import jax
import jax.numpy as jnp
from jax import lax
from jax.experimental import pallas as pl
from jax.experimental.pallas import tpu as pltpu
from jax.experimental.pallas import tpu_sc as plsc

N = 50000
H = 64
E = 800000
NTILE = 800
NSWEEP = NTILE * 32
NPASS = 2
NPAD = NSWEEP * NPASS
CHUNK = 1600
NCHUNKS = E // CHUNK
GRP = CHUNK // 16
BATCH = 64
CAP = CHUNK + BATCH + 32
RB = 2000
NB = N // RB
NEG = float(-3.0e38)


def _lk(x):
    return jnp.where(x >= 0, x, 0.1 * x)


def _dot(a, b):
    return jax.lax.dot_general(a, b, (((1,), (0,)), ((), ())))


def _res(x, w1, b1, w2, b2):
    return x + _dot(_lk(_dot(x, w1) + b1), w2) + b2


_sc_mesh = plsc.VectorSubcoreMesh(core_axis_name="c", subcore_axis_name="s")


@functools.partial(
    pl.kernel,
    mesh=_sc_mesh,
    compiler_params=pltpu.CompilerParams(needs_layout_passes=False, use_tc_tiling_on_sc=False),
    out_type=[
        pltpu.HBM((NPAD, H), jnp.float32),
        pltpu.HBM((NPAD, H), jnp.float32),
        pltpu.HBM((NPAD, 16), jnp.float32),
    ],
    scratch_types=[
        pltpu.VMEM((NTILE, H), jnp.float32),
        pltpu.VMEM((NTILE, H), jnp.float32),
        pltpu.VMEM((NTILE, 16), jnp.float32),
        pltpu.VMEM((CHUNK,), jnp.int32),
        pltpu.VMEM((CHUNK,), jnp.int32),
        pltpu.VMEM((CAP,), jnp.int32),
        pltpu.VMEM((CAP,), jnp.int32),
        pltpu.VMEM((BATCH, H), jnp.float32),
        pltpu.SemaphoreType.DMA,
    ],
)
def _segment_kernel(h_hbm, src_hbm, dst_hbm, out_sum, out_max, out_deg,
                    sacc, macc, dacc, dstst, srcst, csrc, cdst, rows, sem):
    c = lax.axis_index("c")
    s = lax.axis_index("s")
    wid = c * 16 + s

    negv = jnp.full((16,), NEG, jnp.float32)
    zv = jnp.zeros((16,), jnp.float32)
    lanes = lax.iota(jnp.int32, 16)
    e0 = jnp.where(lanes == 0, jnp.full((16,), 1.0, jnp.float32), zv)

    def _flush(off, lo):
        for g in range(4):
            csrc[pl.ds(off + g * 16, 16)] = lanes + (g * 16)
            cdst[pl.ds(off + g * 16, 16)] = lanes
        nbatch = (off + BATCH - 1) // BATCH

        def _batch(bi, carry):
            base = bi * BATCH
            gidx = csrc.at[pl.ds(base, BATCH)]
            pltpu.async_copy(h_hbm.at[gidx], rows, sem).wait()
            nreal = jnp.minimum(off - base, BATCH)

            def _edge(i, ecarry):
                ld = cdst[pl.ds(base + i, 16)][0] - lo
                for j in range(4):
                    rj = rows[i, pl.ds(16 * j, 16)]
                    macc[ld, pl.ds(16 * j, 16)] = jnp.maximum(
                        macc[ld, pl.ds(16 * j, 16)], rj)
                    sacc[ld, pl.ds(16 * j, 16)] = (
                        sacc[ld, pl.ds(16 * j, 16)] + rj)
                dacc[ld, pl.ds(0, 16)] = dacc[ld, pl.ds(0, 16)] + e0
                return ecarry
            lax.fori_loop(0, nreal, _edge, 0)
            return carry
        lax.fori_loop(0, nbatch, _batch, 0)

    def _pass(p):
        lo = p * NSWEEP + wid * NTILE

        def _init_acc(r, carry):
            for j in range(4):
                macc[r, pl.ds(16 * j, 16)] = negv
                sacc[r, pl.ds(16 * j, 16)] = zv
            dacc[r, pl.ds(0, 16)] = zv
            return carry
        lax.fori_loop(0, NTILE, _init_acc, 0)

        def _chunk(k, carry):
            pltpu.sync_copy(dst_hbm.at[pl.ds(k * CHUNK, CHUNK)], dstst)
            pltpu.sync_copy(src_hbm.at[pl.ds(k * CHUNK, CHUNK)], srcst)

            def _group(g, off):
                d = dstst[pl.ds(g * 16, 16)]
                lov = jnp.broadcast_to(lo, (16,))
                m = (d >= lov) & (d < lov + NTILE)
                sv = srcst[pl.ds(g * 16, 16)]
                pref = plsc.cumsum(m.astype(jnp.int32))
                pos = jnp.broadcast_to(off, (16,)) + pref - 1
                plsc.store_scatter(csrc, [pos], sv, mask=m)
                plsc.store_scatter(cdst, [pos], d, mask=m)
                return off + pref[15]
            off = lax.fori_loop(0, GRP, _group, 0)
            _flush(off, lo)
            return carry
        lax.fori_loop(0, NCHUNKS, _chunk, 0)

        pltpu.sync_copy(sacc, out_sum.at[pl.ds(lo, NTILE)])
        pltpu.sync_copy(macc, out_max.at[pl.ds(lo, NTILE)])
        pltpu.sync_copy(dacc, out_deg.at[pl.ds(lo, NTILE)])

    for p in range(NPASS):
        _pass(p)


def _stage_a_body(re_ref, pe_ref, nW, nb_, hW, hb_, nm1W, nm1b, nm2W, nm2b,
                  hm1W, hm1b, hm2W, hm2b, pm1W, pm1b, pm2W, pm2b,
                  nf_ref, hp_ref, hs_ref):
    i = pl.program_id(0)
    nf = _dot(re_ref[...], nW[...]) + nb_[...]
    nf = _res(nf, nm1W[...], nm1b[...], nm2W[...], nm2b[...])
    hf = pe_ref[...] * hW[...] + hb_[...]
    hf = _res(hf, hm1W[...], hm1b[...], hm2W[...], hm2b[...])
    hp = _res(hf, pm1W[...], pm1b[...], pm2W[...], pm2b[...])
    nf_ref[...] = nf
    hp_ref[...] = hp
    bs = jnp.sum(hf, axis=0, keepdims=True)

    @pl.when(i == 0)
    def _():
        hs_ref[...] = bs

    @pl.when(i != 0)
    def _():
        hs_ref[...] = hs_ref[...] + bs


def _combine(nfv, hsv, hmv, dgv, rWv, rbv):
    dv = dgv[:, 0:1]
    hmv = jnp.where(dv > 0, hmv, 0.0)
    hmean = hsv / jnp.maximum(dv, 1.0)
    t = (_dot(hsv, rWv[0:64]) + _dot(hmv, rWv[64:128])
         + _dot(hmean, rWv[128:192]) + _dot(nfv, rWv[192:256]) + rbv)
    return nfv + _lk(t)


def _stage_c_body(nf, hs, hm, dg, rW, rb_, cm1W, cm1b, cm2W, cm2b,
                  rf_ref, hc_ref):
    rf = _combine(nf[...], hs[...], hm[...], dg[...], rW[...], rb_[...])
    rf_ref[...] = rf
    hc_ref[...] = _res(rf, cm1W[...], cm1b[...], cm2W[...], cm2b[...])


def _stage_e_body(rf, hs, hm, dg, rW, rb_, hsum, w1, b1, w2, b2, w3, b3,
                  out_ref, acc_ref):
    i = pl.program_id(0)
    rf2 = _combine(rf[...], hs[...], hm[...], dg[...], rW[...], rb_[...])
    bs = jnp.sum(rf2, axis=0, keepdims=True)

    @pl.when(i == 0)
    def _():
        acc_ref[...] = bs

    @pl.when(i != 0)
    def _():
        acc_ref[...] = acc_ref[...] + bs

    @pl.when(i == NB - 1)
    def _():
        mh = hsum[...] * (1.0 / N)
        mr = acc_ref[...] * (1.0 / N)
        w1v = w1[...]
        h1 = _lk(_dot(mh, w1v[0:64]) + _dot(mr, w1v[64:128]) + b1[...])
        h2 = _lk(_dot(h1, w2[...]) + b2[...])
        out_ref[...] = _dot(h2, w3[...]) + b3[...]


def _full(shape):
    return pl.BlockSpec(shape, lambda i: tuple(0 for _ in shape))


def _rows(width):
    return pl.BlockSpec((RB, width), lambda i: (i, 0))


def kernel(router_embed, packet_embed, params, pass_edge_index,
           connect_edge_index):
    p = params
    f32 = jnp.float32

    def w(name):
        return p[name].astype(f32)

    def b2d(name):
        return p[name].astype(f32).reshape(1, -1)

    stage_a = pl.pallas_call(
        _stage_a_body,
        grid=(NB,),
        in_specs=[
            _rows(6), _rows(1),
            _full((6, H)), _full((1, H)), _full((1, H)), _full((1, H)),
            _full((H, H)), _full((1, H)), _full((H, H)), _full((1, H)),
            _full((H, H)), _full((1, H)), _full((H, H)), _full((1, H)),
            _full((H, H)), _full((1, H)), _full((H, H)), _full((1, H)),
        ],
        out_specs=[_rows(H), _rows(H), _full((1, H))],
        out_shape=[
            jax.ShapeDtypeStruct((N, H), f32),
            jax.ShapeDtypeStruct((N, H), f32),
            jax.ShapeDtypeStruct((1, H), f32),
        ],
    )
    nf, hp, hsum = stage_a(
        router_embed.astype(f32), packet_embed.astype(f32),
        w('fg_node_W'), b2d('fg_node_b'), w('fg_hyper_W'), b2d('fg_hyper_b'),
        w('fg_nmlp_W1'), b2d('fg_nmlp_b1'), w('fg_nmlp_W2'), b2d('fg_nmlp_b2'),
        w('fg_hmlp_W1'), b2d('fg_hmlp_b1'), w('fg_hmlp_W2'), b2d('fg_hmlp_b2'),
        w('pass_m_W1'), b2d('pass_m_b1'), w('pass_m_W2'), b2d('pass_m_b2'),
    )

    s1, m1, d1 = _segment_kernel(hp, pass_edge_index[0], pass_edge_index[1])

    stage_c = pl.pallas_call(
        _stage_c_body,
        grid=(NB,),
        in_specs=[
            _rows(H), _rows(H), _rows(H), _rows(16),
            _full((4 * H, H)), _full((1, H)),
            _full((H, H)), _full((1, H)), _full((H, H)), _full((1, H)),
        ],
        out_specs=[_rows(H), _rows(H)],
        out_shape=[
            jax.ShapeDtypeStruct((N, H), f32),
            jax.ShapeDtypeStruct((N, H), f32),
        ],
    )
    rf, hc = stage_c(
        nf, s1, m1, d1,
        w('pass_r_W'), b2d('pass_r_b'),
        w('conn_m_W1'), b2d('conn_m_b1'), w('conn_m_W2'), b2d('conn_m_b2'),
    )

    s2, m2, d2 = _segment_kernel(hc, connect_edge_index[0],
                                 connect_edge_index[1])

    stage_e = pl.pallas_call(
        _stage_e_body,
        grid=(NB,),
        in_specs=[
            _rows(H), _rows(H), _rows(H), _rows(16),
            _full((4 * H, H)), _full((1, H)), _full((1, H)),
            _full((2 * H, H)), _full((1, H)),
            _full((H, H)), _full((1, H)),
            _full((H, 2)), _full((1, 2)),
        ],
        out_specs=[_full((1, 2))],
        out_shape=[jax.ShapeDtypeStruct((1, 2), f32)],
        scratch_shapes=[pltpu.VMEM((1, H), f32)],
    )
    out = stage_e(
        rf, s2, m2, d2,
        w('conn_r_W'), b2d('conn_r_b'), hsum,
        w('head_W1'), b2d('head_b1'), w('head_W2'), b2d('head_b2'),
        w('head_W3'), b2d('head_b3'),
    )
    return out[0].reshape(2)


# speedup vs baseline: 2.0369x; 1.3841x over previous
import functools

import jax
import jax.numpy as jnp
from jax import lax
from jax.experimental import pallas as pl
from jax.experimental.pallas import tpu as pltpu
from jax.experimental.pallas import tpu_sc as plsc

N = 50000
H = 64
E = 800000
NTILE = 800
NSWEEP = NTILE * 32
NPASS = 2
NPAD = NSWEEP * NPASS
CHUNK = 1600
NCHUNKS = E // CHUNK
GRP = CHUNK // 16
BATCH = 48
CAP = CHUNK + BATCH
RB = 2000
NB = N // RB
NEG = float(-3.0e38)


def _lk(x):
    return jnp.where(x >= 0, x, 0.1 * x)


def _dot(a, b):
    return jax.lax.dot_general(a, b, (((1,), (0,)), ((), ())))


def _res(x, w1, b1, w2, b2):
    return x + _dot(_lk(_dot(x, w1) + b1), w2) + b2


_sc_mesh = plsc.VectorSubcoreMesh(core_axis_name="c", subcore_axis_name="s")


@functools.partial(
    pl.kernel,
    mesh=_sc_mesh,
    compiler_params=pltpu.CompilerParams(needs_layout_passes=False,
                                         use_tc_tiling_on_sc=False),
    out_type=[
        pltpu.HBM((NPAD, H), jnp.float32),
        pltpu.HBM((NPAD, H), jnp.float32),
        pltpu.HBM((NPAD, 16), jnp.float32),
    ],
    scratch_types=[
        pltpu.VMEM((NTILE, H), jnp.float32),
        pltpu.VMEM((NTILE, H), jnp.float32),
        pltpu.VMEM((NTILE, 16), jnp.float32),
        pltpu.VMEM((CHUNK,), jnp.int32),
        pltpu.VMEM((CHUNK,), jnp.int32),
        pltpu.VMEM((CHUNK,), jnp.int32),
        pltpu.VMEM((CHUNK,), jnp.int32),
        pltpu.VMEM((CAP,), jnp.int32),
        pltpu.VMEM((CAP,), jnp.int32),
        pltpu.VMEM((BATCH, H), jnp.float32),
        pltpu.VMEM((BATCH, H), jnp.float32),
        pltpu.SemaphoreType.DMA,
        pltpu.SemaphoreType.DMA,
        pltpu.SemaphoreType.DMA,
        pltpu.SemaphoreType.DMA,
        pltpu.SemaphoreType.DMA,
        pltpu.SemaphoreType.DMA,
    ],
)
def _segment_kernel(h_hbm, src_hbm, dst_hbm, out_sum, out_max, out_deg,
                    sacc, macc, dacc, dst0, dst1, src0, src1, csrc, cdst,
                    rows0, rows1, sd0, sd1, ss0, ss1, sg0, sg1):
    c = lax.axis_index("c")
    s = lax.axis_index("s")
    wid = c * 16 + s

    negv = jnp.full((16,), NEG, jnp.float32)
    zv = jnp.zeros((16,), jnp.float32)
    lanes = lax.iota(jnp.int32, 16)
    e0 = jnp.where(lanes == 0, jnp.full((16,), 1.0, jnp.float32), zv)

    def _chunk_copies(k, dbuf, sbuf, semd, sems):
        sl = pl.ds(k * CHUNK, CHUNK)
        return (pltpu.make_async_copy(dst_hbm.at[sl], dbuf, semd),
                pltpu.make_async_copy(src_hbm.at[sl], sbuf, sems))

    def _start_chunk(k, dbuf, sbuf, semd, sems):
        a, b = _chunk_copies(k, dbuf, sbuf, semd, sems)
        a.start()
        b.start()

    def _wait_chunk(k, dbuf, sbuf, semd, sems):
        a, b = _chunk_copies(k, dbuf, sbuf, semd, sems)
        a.wait()
        b.wait()

    def _gather_copy(b, rowsbuf, sem):
        gidx = csrc.at[pl.ds(b * BATCH, BATCH)]
        return pltpu.make_async_copy(h_hbm.at[gidx], rowsbuf, sem)

    def _scan(dbuf, sbuf, lo):
        lov = jnp.broadcast_to(lo, (16,))

        def _group(g, off):
            d = dbuf[pl.ds(g * 16, 16)]
            m = (d >= lov) & (d < lov + NTILE)
            sv = sbuf[pl.ds(g * 16, 16)]
            pref = plsc.cumsum(m.astype(jnp.int32))
            pos = jnp.broadcast_to(off, (16,)) + pref - 1
            plsc.store_scatter(csrc, [pos], sv, mask=m)
            plsc.store_scatter(cdst, [pos], d, mask=m)
            return off + pref[15]
        return lax.fori_loop(0, GRP, _group, 0)

    def _process(bi, rowsbuf, off, lo):
        base = bi * BATCH
        nreal = jnp.minimum(off - base, BATCH)

        def _edge(i, ecarry):
            ld = cdst[pl.ds(base + i, 16)][0] - lo
            for j in range(4):
                rj = rowsbuf[i, pl.ds(16 * j, 16)]
                macc[ld, pl.ds(16 * j, 16)] = jnp.maximum(
                    macc[ld, pl.ds(16 * j, 16)], rj)
                sacc[ld, pl.ds(16 * j, 16)] = (
                    sacc[ld, pl.ds(16 * j, 16)] + rj)
            dacc[ld, pl.ds(0, 16)] = dacc[ld, pl.ds(0, 16)] + e0
            return ecarry
        lax.fori_loop(0, nreal, _edge, 0)

    def _flush(off, lo):
        for g in range(BATCH // 16):
            csrc[pl.ds(off + g * 16, 16)] = lanes + (g * 16)
            cdst[pl.ds(off + g * 16, 16)] = lanes
        nb = (off + BATCH - 1) // BATCH

        @pl.when(nb > 0)
        def _():
            _gather_copy(0, rows0, sg0).start()

        def _b2(bb, carry):
            b0 = 2 * bb
            b1 = b0 + 1
            _gather_copy(b0, rows0, sg0).wait()

            @pl.when(b1 < nb)
            def _():
                _gather_copy(b1, rows1, sg1).start()
            _process(b0, rows0, off, lo)

            @pl.when(b1 < nb)
            def _():
                _gather_copy(b1, rows1, sg1).wait()

                @pl.when(b0 + 2 < nb)
                def _():
                    _gather_copy(b0 + 2, rows0, sg0).start()
                _process(b1, rows1, off, lo)
            return carry
        lax.fori_loop(0, (nb + 1) // 2, _b2, 0)

    def _pass(p):
        lo = p * NSWEEP + wid * NTILE

        def _init_acc(r, carry):
            for j in range(4):
                macc[r, pl.ds(16 * j, 16)] = negv
                sacc[r, pl.ds(16 * j, 16)] = zv
            dacc[r, pl.ds(0, 16)] = zv
            return carry
        lax.fori_loop(0, NTILE, _init_acc, 0)

        _start_chunk(0, dst0, src0, sd0, ss0)

        def _c2(kk, carry):
            k0 = 2 * kk
            k1 = k0 + 1
            _wait_chunk(k0, dst0, src0, sd0, ss0)
            _start_chunk(k1, dst1, src1, sd1, ss1)
            off = _scan(dst0, src0, lo)
            _flush(off, lo)
            _wait_chunk(k1, dst1, src1, sd1, ss1)

            @pl.when(k0 + 2 < NCHUNKS)
            def _():
                _start_chunk(k0 + 2, dst0, src0, sd0, ss0)
            off1 = _scan(dst1, src1, lo)
            _flush(off1, lo)
            return carry
        lax.fori_loop(0, NCHUNKS // 2, _c2, 0)

        pltpu.sync_copy(sacc, out_sum.at[pl.ds(lo, NTILE)])
        pltpu.sync_copy(macc, out_max.at[pl.ds(lo, NTILE)])
        pltpu.sync_copy(dacc, out_deg.at[pl.ds(lo, NTILE)])

    for p in range(NPASS):
        _pass(p)


def _stage_a_body(re_ref, pe_ref, nW, nb_, hW, hb_, nm1W, nm1b, nm2W, nm2b,
                  hm1W, hm1b, hm2W, hm2b, pm1W, pm1b, pm2W, pm2b,
                  nf_ref, hp_ref, hs_ref):
    i = pl.program_id(0)
    nf = _dot(re_ref[...], nW[...]) + nb_[...]
    nf = _res(nf, nm1W[...], nm1b[...], nm2W[...], nm2b[...])
    hf = pe_ref[...] * hW[...] + hb_[...]
    hf = _res(hf, hm1W[...], hm1b[...], hm2W[...], hm2b[...])
    hp = _res(hf, pm1W[...], pm1b[...], pm2W[...], pm2b[...])
    nf_ref[...] = nf
    hp_ref[...] = hp
    bs = jnp.sum(hf, axis=0, keepdims=True)

    @pl.when(i == 0)
    def _():
        hs_ref[...] = bs

    @pl.when(i != 0)
    def _():
        hs_ref[...] = hs_ref[...] + bs


def _combine(nfv, hsv, hmv, dgv, rWv, rbv):
    dv = dgv[:, 0:1]
    hmv = jnp.where(dv > 0, hmv, 0.0)
    hmean = hsv / jnp.maximum(dv, 1.0)
    t = (_dot(hsv, rWv[0:64]) + _dot(hmv, rWv[64:128])
         + _dot(hmean, rWv[128:192]) + _dot(nfv, rWv[192:256]) + rbv)
    return nfv + _lk(t)


def _stage_c_body(nf, hs, hm, dg, rW, rb_, cm1W, cm1b, cm2W, cm2b,
                  rf_ref, hc_ref):
    rf = _combine(nf[...], hs[...], hm[...], dg[...], rW[...], rb_[...])
    rf_ref[...] = rf
    hc_ref[...] = _res(rf, cm1W[...], cm1b[...], cm2W[...], cm2b[...])


def _stage_e_body(rf, hs, hm, dg, rW, rb_, hsum, w1, b1, w2, b2, w3, b3,
                  out_ref, acc_ref):
    i = pl.program_id(0)
    rf2 = _combine(rf[...], hs[...], hm[...], dg[...], rW[...], rb_[...])
    bs = jnp.sum(rf2, axis=0, keepdims=True)

    @pl.when(i == 0)
    def _():
        acc_ref[...] = bs

    @pl.when(i != 0)
    def _():
        acc_ref[...] = acc_ref[...] + bs

    @pl.when(i == NB - 1)
    def _():
        mh = hsum[...] * (1.0 / N)
        mr = acc_ref[...] * (1.0 / N)
        w1v = w1[...]
        h1 = _lk(_dot(mh, w1v[0:64]) + _dot(mr, w1v[64:128]) + b1[...])
        h2 = _lk(_dot(h1, w2[...]) + b2[...])
        out_ref[...] = _dot(h2, w3[...]) + b3[...]


def _full(shape):
    return pl.BlockSpec(shape, lambda i: tuple(0 for _ in shape))


def _rows(width):
    return pl.BlockSpec((RB, width), lambda i: (i, 0))


def kernel(router_embed, packet_embed, params, pass_edge_index,
           connect_edge_index):
    p = params
    f32 = jnp.float32

    def w(name):
        return p[name].astype(f32)

    def b2d(name):
        return p[name].astype(f32).reshape(1, -1)

    stage_a = pl.pallas_call(
        _stage_a_body,
        grid=(NB,),
        in_specs=[
            _rows(6), _rows(1),
            _full((6, H)), _full((1, H)), _full((1, H)), _full((1, H)),
            _full((H, H)), _full((1, H)), _full((H, H)), _full((1, H)),
            _full((H, H)), _full((1, H)), _full((H, H)), _full((1, H)),
            _full((H, H)), _full((1, H)), _full((H, H)), _full((1, H)),
        ],
        out_specs=[_rows(H), _rows(H), _full((1, H))],
        out_shape=[
            jax.ShapeDtypeStruct((N, H), f32),
            jax.ShapeDtypeStruct((N, H), f32),
            jax.ShapeDtypeStruct((1, H), f32),
        ],
    )
    nf, hp, hsum = stage_a(
        router_embed.astype(f32), packet_embed.astype(f32),
        w('fg_node_W'), b2d('fg_node_b'), w('fg_hyper_W'), b2d('fg_hyper_b'),
        w('fg_nmlp_W1'), b2d('fg_nmlp_b1'), w('fg_nmlp_W2'), b2d('fg_nmlp_b2'),
        w('fg_hmlp_W1'), b2d('fg_hmlp_b1'), w('fg_hmlp_W2'), b2d('fg_hmlp_b2'),
        w('pass_m_W1'), b2d('pass_m_b1'), w('pass_m_W2'), b2d('pass_m_b2'),
    )

    s1, m1, d1 = _segment_kernel(hp, pass_edge_index[0], pass_edge_index[1])

    stage_c = pl.pallas_call(
        _stage_c_body,
        grid=(NB,),
        in_specs=[
            _rows(H), _rows(H), _rows(H), _rows(16),
            _full((4 * H, H)), _full((1, H)),
            _full((H, H)), _full((1, H)), _full((H, H)), _full((1, H)),
        ],
        out_specs=[_rows(H), _rows(H)],
        out_shape=[
            jax.ShapeDtypeStruct((N, H), f32),
            jax.ShapeDtypeStruct((N, H), f32),
        ],
    )
    rf, hc = stage_c(
        nf, s1, m1, d1,
        w('pass_r_W'), b2d('pass_r_b'),
        w('conn_m_W1'), b2d('conn_m_b1'), w('conn_m_W2'), b2d('conn_m_b2'),
    )

    s2, m2, d2 = _segment_kernel(hc, connect_edge_index[0],
                                 connect_edge_index[1])

    stage_e = pl.pallas_call(
        _stage_e_body,
        grid=(NB,),
        in_specs=[
            _rows(H), _rows(H), _rows(H), _rows(16),
            _full((4 * H, H)), _full((1, H)), _full((1, H)),
            _full((2 * H, H)), _full((1, H)),
            _full((H, H)), _full((1, H)),
            _full((H, 2)), _full((1, 2)),
        ],
        out_specs=[_full((1, 2))],
        out_shape=[jax.ShapeDtypeStruct((1, 2), f32)],
        scratch_shapes=[pltpu.VMEM((1, H), f32)],
    )
    out = stage_e(
        rf, s2, m2, d2,
        w('conn_r_W'), b2d('conn_r_b'), hsum,
        w('head_W1'), b2d('head_b1'), w('head_W2'), b2d('head_b2'),
        w('head_W3'), b2d('head_b3'),
    )
    return out[0].reshape(2)
